# Initial kernel scaffold; baseline (speedup 1.0000x reference)
import jax
import jax.numpy as jnp
from jax.experimental import pallas as pl


def kernel(features, edge_index, edge_vals, idx, W_node0, b_node0, W_neigh0, b_neigh0, bias0, gamma0, beta0, mean0, var0, W_node1, W_neigh1, bias1, gamma1, beta1, mean1, var1, W_node_last, W_neigh_last, bias_last):
    raise NotImplementedError("write your pallas kernel here")


# speedup vs baseline: 5.8699x; 5.8699x over previous
import functools

import jax
import jax.numpy as jnp
from jax import lax
from jax.experimental import pallas as pl
from jax.experimental.pallas import tpu as pltpu
from jax.experimental.pallas import tpu_sc as plsc

N = 10000
E = 320000
D = 128
H = 128
C = 40
B = 1024
CP = 128
ROWS = 1000
NC = 2
NS = 16
NW = NC * NS
LANES = 16


def _dense0_body(x_ref, wn_ref, bn_ref, wg_ref, bg_ref, on_ref, og_ref):
    x = x_ref[...]
    on_ref[...] = jnp.dot(x, wn_ref[...], preferred_element_type=jnp.float32) + bn_ref[...]
    og_ref[...] = jnp.dot(x, wg_ref[...], preferred_element_type=jnp.float32) + bg_ref[...]


def _dense0(x, wn, bn, wg, bg):
    grid = (N // ROWS,)
    return pl.pallas_call(
        _dense0_body,
        grid=grid,
        in_specs=[
            pl.BlockSpec((ROWS, D), lambda i: (i, 0)),
            pl.BlockSpec((D, H), lambda i: (0, 0)),
            pl.BlockSpec((1, H), lambda i: (0, 0)),
            pl.BlockSpec((D, H), lambda i: (0, 0)),
            pl.BlockSpec((1, H), lambda i: (0, 0)),
        ],
        out_specs=[
            pl.BlockSpec((ROWS, H), lambda i: (i, 0)),
            pl.BlockSpec((ROWS, H), lambda i: (i, 0)),
        ],
        out_shape=[
            jax.ShapeDtypeStruct((N, H), jnp.float32),
            jax.ShapeDtypeStruct((N, H), jnp.float32),
        ],
    )(x, wn, bn, wg, bg)


def _mid_body(out_w, xn_ref, agg_ref, bias_ref, gamma_ref, beta_ref, mean_ref,
              var_ref, w1_ref, w2_ref, ob_ref, o1_ref, o2_ref):
    pre = xn_ref[...] + agg_ref[0] + agg_ref[1] + bias_ref[...]
    xhat = (pre - mean_ref[...]) * lax.rsqrt(var_ref[...] + 1e-5)
    x = jnp.maximum(xhat * gamma_ref[...] + beta_ref[...], 0.0)
    o1_ref[...] = jnp.dot(x, w1_ref[...], preferred_element_type=jnp.float32) + ob_ref[...]
    o2_ref[...] = jnp.dot(x, w2_ref[...], preferred_element_type=jnp.float32)


def _dense_mid(xn, agg, bias, gamma, beta, mean, var, w1, w2, ob, out_w):
    grid = (N // ROWS,)
    vec = lambda i: (0, 0)
    return pl.pallas_call(
        functools.partial(_mid_body, out_w),
        grid=grid,
        in_specs=[
            pl.BlockSpec((ROWS, H), lambda i: (i, 0)),
            pl.BlockSpec((2, ROWS, H), lambda i: (0, i, 0)),
            pl.BlockSpec((1, H), vec),
            pl.BlockSpec((1, H), vec),
            pl.BlockSpec((1, H), vec),
            pl.BlockSpec((1, H), vec),
            pl.BlockSpec((1, H), vec),
            pl.BlockSpec((H, out_w), vec),
            pl.BlockSpec((H, out_w), vec),
            pl.BlockSpec((1, out_w), vec),
        ],
        out_specs=[
            pl.BlockSpec((ROWS, out_w), lambda i: (i, 0)),
            pl.BlockSpec((ROWS, out_w), lambda i: (i, 0)),
        ],
        out_shape=[
            jax.ShapeDtypeStruct((N, out_w), jnp.float32),
            jax.ShapeDtypeStruct((N, out_w), jnp.float32),
        ],
    )(xn, agg, bias, gamma, beta, mean, var, w1, w2, ob)


def _spmm_sc(xg, src, dst, vals, zeros, width, chunk):
    e_per_w = E // NW
    n_per_s = N // NS
    num_chunks = e_per_w // chunk
    jcount = width // LANES
    ngroups = (chunk + LANES - 1) // LANES
    cpad = ngroups * LANES
    mesh = plsc.VectorSubcoreMesh(core_axis_name="c", subcore_axis_name="s")

    @functools.partial(
        pl.kernel,
        out_type=jax.ShapeDtypeStruct((NC, N, width), jnp.float32),
        mesh=mesh,
        scratch_types=[
            pltpu.VMEM((chunk,), jnp.int32),
            pltpu.VMEM((chunk,), jnp.int32),
            pltpu.VMEM((cpad,), jnp.float32),
            pltpu.VMEM((cpad, width), jnp.float32),
            pltpu.VMEM((LANES,), jnp.float32),
            pltpu.VMEM_SHARED((N, width), jnp.float32),
            pltpu.SemaphoreType.DMA,
        ],
    )
    def k(xg_hbm, src_hbm, dst_hbm, vals_hbm, zeros_hbm, out_hbm,
          srcv, dstv, valsv, rowsv, fencev, agg_sh, sem):
        c = lax.axis_index("c")
        s = lax.axis_index("s")
        w = c * NS + s

        @pl.when(s == 0)
        def _():
            pltpu.sync_copy(zeros_hbm, agg_sh)

        plsc.subcore_barrier()

        def chunk_body(i, carry):
            base = w * e_per_w + i * chunk
            pltpu.sync_copy(src_hbm.at[pl.ds(base, chunk)], srcv)
            pltpu.sync_copy(dst_hbm.at[pl.ds(base, chunk)], dstv)
            pltpu.sync_copy(vals_hbm.at[pl.ds(base, chunk)],
                            valsv.at[pl.ds(0, chunk)])
            pltpu.async_copy(xg_hbm.at[srcv], rowsv.at[pl.ds(0, chunk)],
                             sem).wait()

            def group_body(g, carry2):
                vv = valsv[pl.ds(g * LANES, LANES)]
                for l in range(LANES):
                    vb = vv.at[jnp.full((LANES,), l, jnp.int32)].get(
                        mode="promise_in_bounds")
                    e = g * LANES + l
                    for j in range(jcount):
                        sl = pl.ds(j * LANES, LANES)
                        rowsv[e, sl] = rowsv[e, sl] * vb
                return carry2

            lax.fori_loop(0, ngroups, group_body, 0)
            last = pl.ds((jcount - 1) * LANES, LANES)
            fencev[...] = rowsv[cpad - 1, last]
            pltpu.sync_copy(rowsv.at[pl.ds(0, chunk)], agg_sh.at[dstv],
                            add=True)
            return carry

        lax.fori_loop(0, num_chunks, chunk_body, 0)
        plsc.subcore_barrier()

        @pl.when(s == 0)
        def _():
            pltpu.sync_copy(agg_sh, out_hbm.at[c])

    return k(xg, src, dst, vals, zeros)


def _final_gather(xn, agg, idx):
    b_per_w = B // NW
    jcount = CP // LANES
    mesh = plsc.VectorSubcoreMesh(core_axis_name="c", subcore_axis_name="s")

    @functools.partial(
        pl.kernel,
        out_type=jax.ShapeDtypeStruct((B, CP), jnp.float32),
        mesh=mesh,
        scratch_types=[
            pltpu.VMEM((b_per_w,), jnp.int32),
            pltpu.VMEM((b_per_w, CP), jnp.float32),
            pltpu.VMEM((b_per_w, CP), jnp.float32),
            pltpu.SemaphoreType.DMA,
        ],
    )
    def k(xn_hbm, agg_hbm, idx_hbm, out_hbm, idxv, acc, buf, sem):
        c = lax.axis_index("c")
        s = lax.axis_index("s")
        base = (c * NS + s) * b_per_w
        pltpu.sync_copy(idx_hbm.at[pl.ds(base, b_per_w)], idxv)
        pltpu.async_copy(xn_hbm.at[idxv], acc, sem).wait()
        for part in range(NC):
            pltpu.async_copy(agg_hbm.at[part].at[idxv], buf, sem).wait()
            for r in range(b_per_w):
                for j in range(jcount):
                    sl = pl.ds(j * LANES, LANES)
                    acc[r, sl] = acc[r, sl] + buf[r, sl]
        pltpu.sync_copy(acc, out_hbm.at[pl.ds(base, b_per_w)])

    return k(xn, agg, idx)


def kernel(features, edge_index, edge_vals, idx, W_node0, b_node0, W_neigh0,
           b_neigh0, bias0, gamma0, beta0, mean0, var0, W_node1, W_neigh1,
           bias1, gamma1, beta1, mean1, var1, W_node_last, W_neigh_last,
           bias_last):
    dst = edge_index[0]
    src = edge_index[1]
    row = lambda v: v.reshape(1, -1)
    zeros_h = jnp.zeros((N, H), jnp.float32)
    wnl = jnp.pad(W_node_last, ((0, 0), (0, CP - C)))
    wgl = jnp.pad(W_neigh_last, ((0, 0), (0, CP - C)))
    bl = jnp.pad(bias_last, (0, CP - C))

    xn0, xg0 = _dense0(features, W_node0, row(b_node0), W_neigh0, row(b_neigh0))
    agg0 = _spmm_sc(xg0, src, dst, edge_vals, zeros_h, H, 200)
    xn1, xg1 = _dense_mid(xn0, agg0, row(bias0), row(gamma0), row(beta0),
                          row(mean0), row(var0), W_node1, W_neigh1,
                          row(jnp.zeros((H,), jnp.float32)), H)
    agg1 = _spmm_sc(xg1, src, dst, edge_vals, zeros_h, H, 200)
    xnl, xgl = _dense_mid(xn1, agg1, row(bias1), row(gamma1), row(beta1),
                          row(mean1), row(var1), wnl, wgl, row(bl), CP)
    aggl = _spmm_sc(xgl, src, dst, edge_vals, zeros_h, CP, 200)
    out = _final_gather(xnl, aggl, idx)
    return out[:, :C]


# speedup vs baseline: 7.8609x; 1.3392x over previous
import functools

import jax
import jax.numpy as jnp
from jax import lax
from jax.experimental import pallas as pl
from jax.experimental.pallas import tpu as pltpu
from jax.experimental.pallas import tpu_sc as plsc

N = 10000
E = 320000
D = 128
H = 128
C = 40
B = 1024
CP = 128
ROWS = 1000
NC = 2
NS = 16
NW = NC * NS
LANES = 16


def _dense0_body(x_ref, wn_ref, bn_ref, wg_ref, bg_ref, on_ref, og_ref):
    x = x_ref[...]
    on_ref[...] = jnp.dot(x, wn_ref[...], preferred_element_type=jnp.float32) + bn_ref[...]
    og_ref[...] = jnp.dot(x, wg_ref[...], preferred_element_type=jnp.float32) + bg_ref[...]


def _dense0(x, wn, bn, wg, bg):
    grid = (N // ROWS,)
    return pl.pallas_call(
        _dense0_body,
        grid=grid,
        in_specs=[
            pl.BlockSpec((ROWS, D), lambda i: (i, 0)),
            pl.BlockSpec((D, H), lambda i: (0, 0)),
            pl.BlockSpec((1, H), lambda i: (0, 0)),
            pl.BlockSpec((D, H), lambda i: (0, 0)),
            pl.BlockSpec((1, H), lambda i: (0, 0)),
        ],
        out_specs=[
            pl.BlockSpec((ROWS, H), lambda i: (i, 0)),
            pl.BlockSpec((ROWS, H), lambda i: (i, 0)),
        ],
        out_shape=[
            jax.ShapeDtypeStruct((N, H), jnp.float32),
            jax.ShapeDtypeStruct((N, H), jnp.float32),
        ],
    )(x, wn, bn, wg, bg)


def _mid_body(out_w, xn_ref, agg_ref, bias_ref, gamma_ref, beta_ref, mean_ref,
              var_ref, w1_ref, w2_ref, ob_ref, o1_ref, o2_ref):
    pre = xn_ref[...] + agg_ref[0] + agg_ref[1] + bias_ref[...]
    xhat = (pre - mean_ref[...]) * lax.rsqrt(var_ref[...] + 1e-5)
    x = jnp.maximum(xhat * gamma_ref[...] + beta_ref[...], 0.0)
    o1_ref[...] = jnp.dot(x, w1_ref[...], preferred_element_type=jnp.float32) + ob_ref[...]
    o2_ref[...] = jnp.dot(x, w2_ref[...], preferred_element_type=jnp.float32)


def _dense_mid(xn, agg, bias, gamma, beta, mean, var, w1, w2, ob, out_w):
    grid = (N // ROWS,)
    vec = lambda i: (0, 0)
    return pl.pallas_call(
        functools.partial(_mid_body, out_w),
        grid=grid,
        in_specs=[
            pl.BlockSpec((ROWS, H), lambda i: (i, 0)),
            pl.BlockSpec((2, ROWS, H), lambda i: (0, i, 0)),
            pl.BlockSpec((1, H), vec),
            pl.BlockSpec((1, H), vec),
            pl.BlockSpec((1, H), vec),
            pl.BlockSpec((1, H), vec),
            pl.BlockSpec((1, H), vec),
            pl.BlockSpec((H, out_w), vec),
            pl.BlockSpec((H, out_w), vec),
            pl.BlockSpec((1, out_w), vec),
        ],
        out_specs=[
            pl.BlockSpec((ROWS, out_w), lambda i: (i, 0)),
            pl.BlockSpec((ROWS, out_w), lambda i: (i, 0)),
        ],
        out_shape=[
            jax.ShapeDtypeStruct((N, out_w), jnp.float32),
            jax.ShapeDtypeStruct((N, out_w), jnp.float32),
        ],
    )(xn, agg, bias, gamma, beta, mean, var, w1, w2, ob)


CHUNK = 80
NBUF = 4
T = E // CHUNK
TPW = T // NW
RSLC = 624


def _spmm_sc(xg, packed, vals, zeros, width):
    jcount = width // LANES
    ngroups = CHUNK // LANES
    mesh = plsc.VectorSubcoreMesh(core_axis_name="c", subcore_axis_name="s")

    @functools.partial(
        pl.kernel,
        out_type=jax.ShapeDtypeStruct((NC, N, width), jnp.float32),
        mesh=mesh,
        scratch_types=(
            [pltpu.VMEM((2, CHUNK), jnp.int32)] * NBUF
            + [pltpu.VMEM((CHUNK,), jnp.float32)] * NBUF
            + [pltpu.VMEM((CHUNK, width), jnp.float32)] * NBUF
            + [pltpu.VMEM((LANES,), jnp.float32),
               pltpu.VMEM_SHARED((N, width), jnp.float32)]
            + [pltpu.SemaphoreType.DMA] * (2 * NBUF)
        ),
    )
    def k(xg_hbm, packed_hbm, vals_hbm, zeros_hbm, out_hbm,
          pk0, pk1, pk2, pk3, pv0, pv1, pv2, pv3,
          rw0, rw1, rw2, rw3, fencev, agg_sh,
          g0, g1, g2, g3, s0, s1, s2, s3):
        pk = [pk0, pk1, pk2, pk3]
        pv = [pv0, pv1, pv2, pv3]
        rows = [rw0, rw1, rw2, rw3]
        gsem = [g0, g1, g2, g3]
        ssem = [s0, s1, s2, s3]
        c = lax.axis_index("c")
        s = lax.axis_index("s")
        w = c * NS + s
        t0 = w * TPW

        @pl.when(s < NS - 1)
        def _():
            pltpu.sync_copy(zeros_hbm.at[pl.ds(s * RSLC, RSLC)],
                            agg_sh.at[pl.ds(s * RSLC, RSLC)])

        @pl.when(s == NS - 1)
        def _():
            tail = N - (NS - 1) * RSLC
            pltpu.sync_copy(zeros_hbm.at[pl.ds((NS - 1) * RSLC, tail)],
                            agg_sh.at[pl.ds((NS - 1) * RSLC, tail)])

        plsc.subcore_barrier()

        def launch(i, b):
            pltpu.sync_copy(packed_hbm.at[t0 + i], pk[b])
            pltpu.sync_copy(vals_hbm.at[t0 + i], pv[b])
            pltpu.async_copy(xg_hbm.at[pk[b].at[0]], rows[b], gsem[b])

        def wait_gather(b):
            pltpu.make_async_copy(xg_hbm.at[pk[b].at[0]], rows[b],
                                  gsem[b]).wait()

        def scatter_start(b):
            pltpu.async_copy(rows[b], agg_sh.at[pk[b].at[1]], ssem[b],
                             add=True)

        def wait_scatter(b):
            pltpu.make_async_copy(rows[b], agg_sh.at[pk[b].at[1]],
                                  ssem[b]).wait()

        def scale(b):
            def group_body(g, carry):
                vv = pv[b][pl.ds(g * LANES, LANES)]
                for l in range(LANES):
                    vb = vv.at[jnp.full((LANES,), l, jnp.int32)].get(
                        mode="promise_in_bounds")
                    e = g * LANES + l
                    for j in range(jcount):
                        sl = pl.ds(j * LANES, LANES)
                        rows[b][e, sl] = rows[b][e, sl] * vb
                return carry

            lax.fori_loop(0, ngroups, group_body, 0)
            fencev[...] = rows[b][CHUNK - 1,
                                  pl.ds((jcount - 1) * LANES, LANES)]

        launch(0, 0)
        launch(1, 1)

        def pipe_body(p, carry):
            for j in range(NBUF):
                i = p * NBUF + j

                @pl.when(i >= 2)
                def _():
                    wait_scatter((j + 2) % NBUF)

                @pl.when(i + 2 <= TPW - 1)
                def _():
                    launch(i + 2, (j + 2) % NBUF)

                wait_gather(j)
                scale(j)
                scatter_start(j)
            return carry

        lax.fori_loop(0, (TPW - 1) // NBUF, pipe_body, 0)
        wait_gather(0)
        scale(0)
        scatter_start(0)
        wait_scatter(2)
        wait_scatter(3)
        wait_scatter(0)
        plsc.subcore_barrier()

        @pl.when(s < NS - 1)
        def _():
            pltpu.sync_copy(agg_sh.at[pl.ds(s * RSLC, RSLC)],
                            out_hbm.at[c, pl.ds(s * RSLC, RSLC)])

        @pl.when(s == NS - 1)
        def _():
            tail = N - (NS - 1) * RSLC
            pltpu.sync_copy(agg_sh.at[pl.ds((NS - 1) * RSLC, tail)],
                            out_hbm.at[c, pl.ds((NS - 1) * RSLC, tail)])

    return k(xg, packed, vals, zeros)


def _final_gather(xn, agg, idx):
    b_per_w = B // NW
    jcount = CP // LANES
    mesh = plsc.VectorSubcoreMesh(core_axis_name="c", subcore_axis_name="s")

    @functools.partial(
        pl.kernel,
        out_type=jax.ShapeDtypeStruct((B, CP), jnp.float32),
        mesh=mesh,
        scratch_types=[
            pltpu.VMEM((b_per_w,), jnp.int32),
            pltpu.VMEM((b_per_w, CP), jnp.float32),
            pltpu.VMEM((b_per_w, CP), jnp.float32),
            pltpu.SemaphoreType.DMA,
        ],
    )
    def k(xn_hbm, agg_hbm, idx_hbm, out_hbm, idxv, acc, buf, sem):
        c = lax.axis_index("c")
        s = lax.axis_index("s")
        base = (c * NS + s) * b_per_w
        pltpu.sync_copy(idx_hbm.at[pl.ds(base, b_per_w)], idxv)
        pltpu.async_copy(xn_hbm.at[idxv], acc, sem).wait()
        for part in range(NC):
            pltpu.async_copy(agg_hbm.at[part].at[idxv], buf, sem).wait()
            for r in range(b_per_w):
                for j in range(jcount):
                    sl = pl.ds(j * LANES, LANES)
                    acc[r, sl] = acc[r, sl] + buf[r, sl]
        pltpu.sync_copy(acc, out_hbm.at[pl.ds(base, b_per_w)])

    return k(xn, agg, idx)


def kernel(features, edge_index, edge_vals, idx, W_node0, b_node0, W_neigh0,
           b_neigh0, bias0, gamma0, beta0, mean0, var0, W_node1, W_neigh1,
           bias1, gamma1, beta1, mean1, var1, W_node_last, W_neigh_last,
           bias_last):
    dst = edge_index[0]
    src = edge_index[1]
    packed = jnp.stack([src.reshape(T, CHUNK), dst.reshape(T, CHUNK)], axis=1)
    vals = edge_vals.reshape(T, CHUNK).astype(jnp.float32)
    row = lambda v: v.reshape(1, -1)
    zeros_h = jnp.zeros((N, H), jnp.float32)
    wnl = jnp.pad(W_node_last, ((0, 0), (0, CP - C)))
    wgl = jnp.pad(W_neigh_last, ((0, 0), (0, CP - C)))
    bl = jnp.pad(bias_last, (0, CP - C))

    xn0, xg0 = _dense0(features, W_node0, row(b_node0), W_neigh0, row(b_neigh0))
    agg0 = _spmm_sc(xg0, packed, vals, zeros_h, H)
    xn1, xg1 = _dense_mid(xn0, agg0, row(bias0), row(gamma0), row(beta0),
                          row(mean0), row(var0), W_node1, W_neigh1,
                          row(jnp.zeros((H,), jnp.float32)), H)
    agg1 = _spmm_sc(xg1, packed, vals, zeros_h, H)
    xnl, xgl = _dense_mid(xn1, agg1, row(bias1), row(gamma1), row(beta1),
                          row(mean1), row(var1), wnl, wgl, row(bl), CP)
    aggl = _spmm_sc(xgl, packed, vals, zeros_h, CP)
    out = _final_gather(xnl, aggl, idx)
    return out[:, :C]


# speedup vs baseline: 8.1138x; 1.0322x over previous
import functools

import jax
import jax.numpy as jnp
from jax import lax
from jax.experimental import pallas as pl
from jax.experimental.pallas import tpu as pltpu
from jax.experimental.pallas import tpu_sc as plsc

N = 10000
E = 320000
D = 128
H = 128
C = 40
B = 1024
CP = 128
ROWS = 1000
NC = 2
NS = 16
NW = NC * NS
LANES = 16


def _dense0_body(x_ref, wn_ref, bn_ref, wg_ref, bg_ref, on_ref, og_ref):
    x = x_ref[...]
    on_ref[...] = jnp.dot(x, wn_ref[...], preferred_element_type=jnp.float32) + bn_ref[...]
    og_ref[...] = jnp.dot(x, wg_ref[...], preferred_element_type=jnp.float32) + bg_ref[...]


def _dense0(x, wn, bn, wg, bg):
    grid = (N // ROWS,)
    return pl.pallas_call(
        _dense0_body,
        grid=grid,
        in_specs=[
            pl.BlockSpec((ROWS, D), lambda i: (i, 0)),
            pl.BlockSpec((D, H), lambda i: (0, 0)),
            pl.BlockSpec((1, H), lambda i: (0, 0)),
            pl.BlockSpec((D, H), lambda i: (0, 0)),
            pl.BlockSpec((1, H), lambda i: (0, 0)),
        ],
        out_specs=[
            pl.BlockSpec((ROWS, H), lambda i: (i, 0)),
            pl.BlockSpec((ROWS, H), lambda i: (i, 0)),
        ],
        out_shape=[
            jax.ShapeDtypeStruct((N, H), jnp.float32),
            jax.ShapeDtypeStruct((N, H), jnp.float32),
        ],
    )(x, wn, bn, wg, bg)


def _mid_body(out_w, xn_ref, agg_ref, bias_ref, gamma_ref, beta_ref, mean_ref,
              var_ref, w1_ref, w2_ref, ob_ref, o1_ref, o2_ref):
    pre = xn_ref[...] + agg_ref[0] + agg_ref[1] + bias_ref[...]
    xhat = (pre - mean_ref[...]) * lax.rsqrt(var_ref[...] + 1e-5)
    x = jnp.maximum(xhat * gamma_ref[...] + beta_ref[...], 0.0)
    o1_ref[...] = jnp.dot(x, w1_ref[...], preferred_element_type=jnp.float32) + ob_ref[...]
    o2_ref[...] = jnp.dot(x, w2_ref[...], preferred_element_type=jnp.float32)


def _dense_mid(xn, agg, bias, gamma, beta, mean, var, w1, w2, ob, out_w):
    grid = (N // ROWS,)
    vec = lambda i: (0, 0)
    return pl.pallas_call(
        functools.partial(_mid_body, out_w),
        grid=grid,
        in_specs=[
            pl.BlockSpec((ROWS, H), lambda i: (i, 0)),
            pl.BlockSpec((2, ROWS, H), lambda i: (0, i, 0)),
            pl.BlockSpec((1, H), vec),
            pl.BlockSpec((1, H), vec),
            pl.BlockSpec((1, H), vec),
            pl.BlockSpec((1, H), vec),
            pl.BlockSpec((1, H), vec),
            pl.BlockSpec((H, out_w), vec),
            pl.BlockSpec((H, out_w), vec),
            pl.BlockSpec((1, out_w), vec),
        ],
        out_specs=[
            pl.BlockSpec((ROWS, out_w), lambda i: (i, 0)),
            pl.BlockSpec((ROWS, out_w), lambda i: (i, 0)),
        ],
        out_shape=[
            jax.ShapeDtypeStruct((N, out_w), jnp.float32),
            jax.ShapeDtypeStruct((N, out_w), jnp.float32),
        ],
    )(xn, agg, bias, gamma, beta, mean, var, w1, w2, ob)


CHUNK = 80
NBUF = 4
T = E // CHUNK
TPW = T // NW
RSLC = 624


def _spmm_sc(xg, packed, vals, zeros, width, live):
    jcount = live // LANES
    ngroups = CHUNK // LANES
    mesh = plsc.VectorSubcoreMesh(core_axis_name="c", subcore_axis_name="s")

    @functools.partial(
        pl.kernel,
        out_type=jax.ShapeDtypeStruct((NC, N, width), jnp.float32),
        mesh=mesh,
        scratch_types=(
            [pltpu.VMEM((2, CHUNK), jnp.int32)] * NBUF
            + [pltpu.VMEM((CHUNK,), jnp.float32)] * NBUF
            + [pltpu.VMEM((CHUNK, width), jnp.float32)] * NBUF
            + [pltpu.VMEM((LANES,), jnp.float32),
               pltpu.VMEM_SHARED((N, width), jnp.float32)]
            + [pltpu.SemaphoreType.DMA] * (2 * NBUF)
        ),
    )
    def k(xg_hbm, packed_hbm, vals_hbm, zeros_hbm, out_hbm,
          pk0, pk1, pk2, pk3, pv0, pv1, pv2, pv3,
          rw0, rw1, rw2, rw3, fencev, agg_sh,
          g0, g1, g2, g3, s0, s1, s2, s3):
        pk = [pk0, pk1, pk2, pk3]
        pv = [pv0, pv1, pv2, pv3]
        rows = [rw0, rw1, rw2, rw3]
        gsem = [g0, g1, g2, g3]
        ssem = [s0, s1, s2, s3]
        c = lax.axis_index("c")
        s = lax.axis_index("s")
        w = c * NS + s
        t0 = w * TPW

        @pl.when(s < NS - 1)
        def _():
            pltpu.sync_copy(zeros_hbm.at[pl.ds(s * RSLC, RSLC)],
                            agg_sh.at[pl.ds(s * RSLC, RSLC)])

        @pl.when(s == NS - 1)
        def _():
            tail = N - (NS - 1) * RSLC
            pltpu.sync_copy(zeros_hbm.at[pl.ds((NS - 1) * RSLC, tail)],
                            agg_sh.at[pl.ds((NS - 1) * RSLC, tail)])

        plsc.subcore_barrier()

        def launch(i, b):
            pltpu.sync_copy(packed_hbm.at[t0 + i], pk[b])
            pltpu.sync_copy(vals_hbm.at[t0 + i], pv[b])
            pltpu.async_copy(xg_hbm.at[pk[b].at[0]], rows[b], gsem[b])

        def wait_gather(b):
            pltpu.make_async_copy(xg_hbm.at[pk[b].at[0]], rows[b],
                                  gsem[b]).wait()

        def scatter_start(b):
            pltpu.async_copy(rows[b], agg_sh.at[pk[b].at[1]], ssem[b],
                             add=True)

        def wait_scatter(b):
            pltpu.make_async_copy(rows[b], agg_sh.at[pk[b].at[1]],
                                  ssem[b]).wait()

        def scale(b):
            def group_body(g, carry):
                vv = pv[b][pl.ds(g * LANES, LANES)]
                for l in range(LANES):
                    vb = vv.at[jnp.full((LANES,), l, jnp.int32)].get(
                        mode="promise_in_bounds")
                    e = g * LANES + l
                    for j in range(jcount):
                        sl = pl.ds(j * LANES, LANES)
                        rows[b][e, sl] = rows[b][e, sl] * vb
                return carry

            lax.fori_loop(0, ngroups, group_body, 0)
            fencev[...] = rows[b][CHUNK - 1,
                                  pl.ds((jcount - 1) * LANES, LANES)]

        launch(0, 0)
        launch(1, 1)

        def pipe_body(p, carry):
            for j in range(NBUF):
                i = p * NBUF + j

                @pl.when(i >= 2)
                def _():
                    wait_scatter((j + 2) % NBUF)

                @pl.when(i + 2 <= TPW - 1)
                def _():
                    launch(i + 2, (j + 2) % NBUF)

                wait_gather(j)
                scale(j)
                scatter_start(j)
            return carry

        lax.fori_loop(0, (TPW - 1) // NBUF, pipe_body, 0)
        wait_gather(0)
        scale(0)
        scatter_start(0)
        wait_scatter(2)
        wait_scatter(3)
        wait_scatter(0)
        plsc.subcore_barrier()

        @pl.when(s < NS - 1)
        def _():
            pltpu.sync_copy(agg_sh.at[pl.ds(s * RSLC, RSLC)],
                            out_hbm.at[c, pl.ds(s * RSLC, RSLC)])

        @pl.when(s == NS - 1)
        def _():
            tail = N - (NS - 1) * RSLC
            pltpu.sync_copy(agg_sh.at[pl.ds((NS - 1) * RSLC, tail)],
                            out_hbm.at[c, pl.ds((NS - 1) * RSLC, tail)])

    return k(xg, packed, vals, zeros)


def _final_gather(xn, agg, idx):
    b_per_w = B // NW
    jcount = CP // LANES
    mesh = plsc.VectorSubcoreMesh(core_axis_name="c", subcore_axis_name="s")

    @functools.partial(
        pl.kernel,
        out_type=jax.ShapeDtypeStruct((B, CP), jnp.float32),
        mesh=mesh,
        scratch_types=[
            pltpu.VMEM((b_per_w,), jnp.int32),
            pltpu.VMEM((b_per_w, CP), jnp.float32),
            pltpu.VMEM((b_per_w, CP), jnp.float32),
            pltpu.SemaphoreType.DMA,
        ],
    )
    def k(xn_hbm, agg_hbm, idx_hbm, out_hbm, idxv, acc, buf, sem):
        c = lax.axis_index("c")
        s = lax.axis_index("s")
        base = (c * NS + s) * b_per_w
        pltpu.sync_copy(idx_hbm.at[pl.ds(base, b_per_w)], idxv)
        pltpu.async_copy(xn_hbm.at[idxv], acc, sem).wait()
        for part in range(NC):
            pltpu.async_copy(agg_hbm.at[part].at[idxv], buf, sem).wait()
            for r in range(b_per_w):
                for j in range(jcount):
                    sl = pl.ds(j * LANES, LANES)
                    acc[r, sl] = acc[r, sl] + buf[r, sl]
        pltpu.sync_copy(acc, out_hbm.at[pl.ds(base, b_per_w)])

    return k(xn, agg, idx)


def kernel(features, edge_index, edge_vals, idx, W_node0, b_node0, W_neigh0,
           b_neigh0, bias0, gamma0, beta0, mean0, var0, W_node1, W_neigh1,
           bias1, gamma1, beta1, mean1, var1, W_node_last, W_neigh_last,
           bias_last):
    dst = edge_index[0]
    src = edge_index[1]
    packed = jnp.stack([src.reshape(T, CHUNK), dst.reshape(T, CHUNK)], axis=1)
    vals = edge_vals.reshape(T, CHUNK).astype(jnp.float32)
    row = lambda v: v.reshape(1, -1)
    zeros_h = jnp.zeros((N, H), jnp.float32)
    wnl = jnp.pad(W_node_last, ((0, 0), (0, CP - C)))
    wgl = jnp.pad(W_neigh_last, ((0, 0), (0, CP - C)))
    bl = jnp.pad(bias_last, (0, CP - C))

    xn0, xg0 = _dense0(features, W_node0, row(b_node0), W_neigh0, row(b_neigh0))
    agg0 = _spmm_sc(xg0, packed, vals, zeros_h, H, H)
    xn1, xg1 = _dense_mid(xn0, agg0, row(bias0), row(gamma0), row(beta0),
                          row(mean0), row(var0), W_node1, W_neigh1,
                          row(jnp.zeros((H,), jnp.float32)), H)
    agg1 = _spmm_sc(xg1, packed, vals, zeros_h, H, H)
    xnl, xgl = _dense_mid(xn1, agg1, row(bias1), row(gamma1), row(beta1),
                          row(mean1), row(var1), wnl, wgl, row(bl), CP)
    aggl = _spmm_sc(xgl, packed, vals, zeros_h, CP, 48)
    out = _final_gather(xnl, aggl, idx)
    return out[:, :C]


# speedup vs baseline: 11.9663x; 1.4748x over previous
import functools

import jax
import jax.numpy as jnp
from jax import lax
from jax.experimental import pallas as pl
from jax.experimental.pallas import tpu as pltpu
from jax.experimental.pallas import tpu_sc as plsc

N = 10000
E = 320000
D = 128
H = 128
C = 40
B = 1024
CP = 128
ROWS = 1000
NC = 2
NS = 16
NW = NC * NS
LANES = 16


def _dense0_body(x_ref, wn_ref, bn_ref, wg_ref, bg_ref, on_ref, og_ref):
    x = x_ref[...]
    on_ref[...] = jnp.dot(x, wn_ref[...], preferred_element_type=jnp.float32) + bn_ref[...]
    og_ref[...] = jnp.dot(x, wg_ref[...], preferred_element_type=jnp.float32) + bg_ref[...]


def _dense0(x, wn, bn, wg, bg):
    grid = (N // ROWS,)
    return pl.pallas_call(
        _dense0_body,
        grid=grid,
        in_specs=[
            pl.BlockSpec((ROWS, D), lambda i: (i, 0)),
            pl.BlockSpec((D, H), lambda i: (0, 0)),
            pl.BlockSpec((1, H), lambda i: (0, 0)),
            pl.BlockSpec((D, H), lambda i: (0, 0)),
            pl.BlockSpec((1, H), lambda i: (0, 0)),
        ],
        out_specs=[
            pl.BlockSpec((ROWS, H), lambda i: (i, 0)),
            pl.BlockSpec((ROWS, H), lambda i: (i, 0)),
        ],
        out_shape=[
            jax.ShapeDtypeStruct((N, H), jnp.float32),
            jax.ShapeDtypeStruct((N, H), jnp.float32),
        ],
    )(x, wn, bn, wg, bg)


def _mid_body(out_w, xn_ref, agg_ref, bias_ref, gamma_ref, beta_ref, mean_ref,
              var_ref, w1_ref, w2_ref, ob_ref, o1_ref, o2_ref):
    pre = xn_ref[...] + agg_ref[0] + agg_ref[1] + bias_ref[...]
    xhat = (pre - mean_ref[...]) * lax.rsqrt(var_ref[...] + 1e-5)
    x = jnp.maximum(xhat * gamma_ref[...] + beta_ref[...], 0.0)
    o1_ref[...] = jnp.dot(x, w1_ref[...], preferred_element_type=jnp.float32) + ob_ref[...]
    o2_ref[...] = jnp.dot(x, w2_ref[...], preferred_element_type=jnp.float32)


def _dense_mid(xn, agg, bias, gamma, beta, mean, var, w1, w2, ob, out_w):
    grid = (N // ROWS,)
    vec = lambda i: (0, 0)
    return pl.pallas_call(
        functools.partial(_mid_body, out_w),
        grid=grid,
        in_specs=[
            pl.BlockSpec((ROWS, H), lambda i: (i, 0)),
            pl.BlockSpec((2, ROWS, H), lambda i: (0, i, 0)),
            pl.BlockSpec((1, H), vec),
            pl.BlockSpec((1, H), vec),
            pl.BlockSpec((1, H), vec),
            pl.BlockSpec((1, H), vec),
            pl.BlockSpec((1, H), vec),
            pl.BlockSpec((H, out_w), vec),
            pl.BlockSpec((H, out_w), vec),
            pl.BlockSpec((1, out_w), vec),
        ],
        out_specs=[
            pl.BlockSpec((ROWS, out_w), lambda i: (i, 0)),
            pl.BlockSpec((ROWS, out_w), lambda i: (i, 0)),
        ],
        out_shape=[
            jax.ShapeDtypeStruct((N, out_w), jnp.float32),
            jax.ShapeDtypeStruct((N, out_w), jnp.float32),
        ],
    )(xn, agg, bias, gamma, beta, mean, var, w1, w2, ob)


CHUNK = 80
NBUF = 4
T = E // CHUNK
TPW = T // NW
RSLC = 624


def _spmm_sc(xg, packed, vals, zeros, width, live):
    jcount = live // LANES
    ngroups = CHUNK // LANES
    mesh = plsc.VectorSubcoreMesh(core_axis_name="c", subcore_axis_name="s")

    @functools.partial(
        pl.kernel,
        out_type=jax.ShapeDtypeStruct((NC, N, width), jnp.float32),
        mesh=mesh,
        scratch_types=(
            [pltpu.VMEM((2, CHUNK), jnp.int32)] * NBUF
            + [pltpu.VMEM((CHUNK,), jnp.float32)] * NBUF
            + [pltpu.VMEM((CHUNK, width), jnp.float32)] * NBUF
            + [pltpu.VMEM((LANES,), jnp.float32),
               pltpu.VMEM_SHARED((N, width), jnp.float32)]
            + [pltpu.SemaphoreType.DMA] * (4 * NBUF)
        ),
    )
    def k(xg_hbm, packed_hbm, vals_hbm, zeros_hbm, out_hbm,
          pk0, pk1, pk2, pk3, pv0, pv1, pv2, pv3,
          rw0, rw1, rw2, rw3, fencev, agg_sh,
          g0, g1, g2, g3, s0, s1, s2, s3,
          dk0, dk1, dk2, dk3, dv0, dv1, dv2, dv3):
        pk = [pk0, pk1, pk2, pk3]
        pv = [pv0, pv1, pv2, pv3]
        rows = [rw0, rw1, rw2, rw3]
        gsem = [g0, g1, g2, g3]
        ssem = [s0, s1, s2, s3]
        dksem = [dk0, dk1, dk2, dk3]
        dvsem = [dv0, dv1, dv2, dv3]
        c = lax.axis_index("c")
        s = lax.axis_index("s")
        w = c * NS + s
        t0 = w * TPW

        @pl.when(s < NS - 1)
        def _():
            pltpu.sync_copy(zeros_hbm.at[pl.ds(s * RSLC, RSLC)],
                            agg_sh.at[pl.ds(s * RSLC, RSLC)])

        @pl.when(s == NS - 1)
        def _():
            tail = N - (NS - 1) * RSLC
            pltpu.sync_copy(zeros_hbm.at[pl.ds((NS - 1) * RSLC, tail)],
                            agg_sh.at[pl.ds((NS - 1) * RSLC, tail)])

        plsc.subcore_barrier()

        def fetch(i, b):
            pltpu.async_copy(packed_hbm.at[t0 + i], pk[b], dksem[b])
            pltpu.async_copy(vals_hbm.at[t0 + i], pv[b], dvsem[b])

        def wait_fetch(i, b):
            pltpu.make_async_copy(packed_hbm.at[t0 + i], pk[b],
                                  dksem[b]).wait()
            pltpu.make_async_copy(vals_hbm.at[t0 + i], pv[b],
                                  dvsem[b]).wait()

        def gather(b):
            pltpu.async_copy(xg_hbm.at[pk[b].at[0]], rows[b], gsem[b])

        def wait_gather(b):
            pltpu.make_async_copy(xg_hbm.at[pk[b].at[0]], rows[b],
                                  gsem[b]).wait()

        def scatter_start(b):
            pltpu.async_copy(rows[b], agg_sh.at[pk[b].at[1]], ssem[b],
                             add=True)

        def wait_scatter(b):
            pltpu.make_async_copy(rows[b], agg_sh.at[pk[b].at[1]],
                                  ssem[b]).wait()

        def scale(b):
            def group_body(g, carry):
                vv = pv[b][pl.ds(g * LANES, LANES)]
                for l in range(LANES):
                    vb = vv.at[jnp.full((LANES,), l, jnp.int32)].get(
                        mode="promise_in_bounds")
                    e = g * LANES + l
                    for j in range(jcount):
                        sl = pl.ds(j * LANES, LANES)
                        rows[b][e, sl] = rows[b][e, sl] * vb
                return carry

            lax.fori_loop(0, ngroups, group_body, 0)
            fencev[...] = rows[b][CHUNK - 1,
                                  pl.ds((jcount - 1) * LANES, LANES)]

        fetch(0, 0)
        fetch(1, 1)
        fetch(2, 2)
        wait_fetch(0, 0)
        gather(0)
        wait_fetch(1, 1)
        gather(1)

        def pipe_body(p, carry):
            for j in range(NBUF):
                i = p * NBUF + j

                @pl.when(i >= 1)
                def _():
                    wait_scatter((j + 3) % NBUF)

                @pl.when(i <= TPW - 4)
                def _():
                    fetch(i + 3, (j + 3) % NBUF)

                @pl.when(i <= TPW - 3)
                def _():
                    wait_fetch(i + 2, (j + 2) % NBUF)
                    gather((j + 2) % NBUF)

                wait_gather(j)
                scale(j)
                scatter_start(j)
            return carry

        lax.fori_loop(0, (TPW - 1) // NBUF, pipe_body, 0)
        wait_scatter(3)
        wait_gather(0)
        scale(0)
        scatter_start(0)
        wait_scatter(0)
        plsc.subcore_barrier()

        @pl.when(s < NS - 1)
        def _():
            pltpu.sync_copy(agg_sh.at[pl.ds(s * RSLC, RSLC)],
                            out_hbm.at[c, pl.ds(s * RSLC, RSLC)])

        @pl.when(s == NS - 1)
        def _():
            tail = N - (NS - 1) * RSLC
            pltpu.sync_copy(agg_sh.at[pl.ds((NS - 1) * RSLC, tail)],
                            out_hbm.at[c, pl.ds((NS - 1) * RSLC, tail)])

    return k(xg, packed, vals, zeros)


def _final_gather(xn, agg, idx):
    b_per_w = B // NW
    jcount = CP // LANES
    mesh = plsc.VectorSubcoreMesh(core_axis_name="c", subcore_axis_name="s")

    @functools.partial(
        pl.kernel,
        out_type=jax.ShapeDtypeStruct((B, CP), jnp.float32),
        mesh=mesh,
        scratch_types=[
            pltpu.VMEM((b_per_w,), jnp.int32),
            pltpu.VMEM((b_per_w, CP), jnp.float32),
            pltpu.VMEM((b_per_w, CP), jnp.float32),
            pltpu.SemaphoreType.DMA,
        ],
    )
    def k(xn_hbm, agg_hbm, idx_hbm, out_hbm, idxv, acc, buf, sem):
        c = lax.axis_index("c")
        s = lax.axis_index("s")
        base = (c * NS + s) * b_per_w
        pltpu.sync_copy(idx_hbm.at[pl.ds(base, b_per_w)], idxv)
        pltpu.async_copy(xn_hbm.at[idxv], acc, sem).wait()
        for part in range(NC):
            pltpu.async_copy(agg_hbm.at[part].at[idxv], buf, sem).wait()
            for r in range(b_per_w):
                for j in range(jcount):
                    sl = pl.ds(j * LANES, LANES)
                    acc[r, sl] = acc[r, sl] + buf[r, sl]
        pltpu.sync_copy(acc, out_hbm.at[pl.ds(base, b_per_w)])

    return k(xn, agg, idx)


def kernel(features, edge_index, edge_vals, idx, W_node0, b_node0, W_neigh0,
           b_neigh0, bias0, gamma0, beta0, mean0, var0, W_node1, W_neigh1,
           bias1, gamma1, beta1, mean1, var1, W_node_last, W_neigh_last,
           bias_last):
    dst = edge_index[0]
    src = edge_index[1]
    packed = jnp.stack([src.reshape(T, CHUNK), dst.reshape(T, CHUNK)], axis=1)
    vals = edge_vals.reshape(T, CHUNK).astype(jnp.float32)
    row = lambda v: v.reshape(1, -1)
    zeros_h = jnp.zeros((N, H), jnp.float32)
    wnl = jnp.pad(W_node_last, ((0, 0), (0, CP - C)))
    wgl = jnp.pad(W_neigh_last, ((0, 0), (0, CP - C)))
    bl = jnp.pad(bias_last, (0, CP - C))

    xn0, xg0 = _dense0(features, W_node0, row(b_node0), W_neigh0, row(b_neigh0))
    agg0 = _spmm_sc(xg0, packed, vals, zeros_h, H, H)
    xn1, xg1 = _dense_mid(xn0, agg0, row(bias0), row(gamma0), row(beta0),
                          row(mean0), row(var0), W_node1, W_neigh1,
                          row(jnp.zeros((H,), jnp.float32)), H)
    agg1 = _spmm_sc(xg1, packed, vals, zeros_h, H, H)
    xnl, xgl = _dense_mid(xn1, agg1, row(bias1), row(gamma1), row(beta1),
                          row(mean1), row(var1), wnl, wgl, row(bl), CP)
    aggl = _spmm_sc(xgl, packed, vals, zeros_h, CP, 48)
    out = _final_gather(xnl, aggl, idx)
    return out[:, :C]
